# baseline (speedup 1.0000x reference)
import functools

import jax
import jax.numpy as jnp
from jax import lax
from jax.experimental import pallas as pl
from jax.experimental.pallas import tpu as pltpu
from jax.experimental.pallas import tpu_sc as plsc

_B, _N, _C, _H, _W = 2, 512, 256, 7, 7
_D_IN = _C * _H * _W
_D_HID = 1024
_OUT = 16
_M = _B * _N

_NWORKERS = 32
_ROWS_PER_W = _M // _NWORKERS
_GROUP = 8

_TK = 1792


_CCHUNK = 32


def _sc_compact(aligned_feat):
    mesh = plsc.VectorSubcoreMesh(core_axis_name="c", subcore_axis_name="s")
    n_groups = _ROWS_PER_W // _GROUP
    n_cc = _C // _CCHUNK
    n_dma = n_groups * n_cc * _H

    n_in = n_cc * _H

    @functools.partial(
        pl.kernel,
        mesh=mesh,
        out_type=jax.ShapeDtypeStruct((_M, _C, _H, _W), jnp.float32),
        scratch_types=[
            pltpu.VMEM((_GROUP, _C, _H, _W), jnp.float32),
            pltpu.SemaphoreType.DMA,
            pltpu.SemaphoreType.DMA,
        ],
        compiler_params=pltpu.CompilerParams(use_tc_tiling_on_sc=False),
    )
    def k(feat_hbm, out_hbm, vbuf, isem, osem):
        wid = lax.axis_index("s") * 2 + lax.axis_index("c")
        row0 = wid * _ROWS_PER_W

        def group(g, _):
            m0 = row0 + g * _GROUP

            def in_copy(j):
                m = m0 + j
                return pltpu.make_async_copy(
                    feat_hbm.at[m // _N, m % _N], vbuf.at[j], isem)

            def fire(j, _):
                in_copy(j).start()
                return 0

            lax.fori_loop(0, _GROUP, fire, 0)

            def drain(j, _):
                in_copy(j).wait()
                return 0

            lax.fori_loop(0, _GROUP, drain, 0)

            out = pltpu.make_async_copy(
                vbuf, out_hbm.at[pl.ds(m0, _GROUP)], osem)
            out.start()
            out.wait()
            return 0

        lax.fori_loop(0, _ROWS_PER_W // _GROUP, group, 0)

    return k(aligned_feat)


def _mlp_kernel(x_ref, w1_ref, b1_ref, w2_ref, b2_ref, wh_ref, bh_ref,
                o_ref, acc_ref):
    kk = pl.program_id(1)

    @pl.when(kk == 0)
    def _init():
        acc_ref[...] = jnp.zeros_like(acc_ref)

    acc_ref[...] += jnp.dot(x_ref[...], w1_ref[...],
                            preferred_element_type=jnp.float32)

    @pl.when(kk == pl.num_programs(1) - 1)
    def _finish():
        h1 = jnp.maximum(acc_ref[...] + b1_ref[...], 0.0)
        h2 = jnp.maximum(
            jnp.dot(h1, w2_ref[...], preferred_element_type=jnp.float32)
            + b2_ref[...], 0.0)
        o_ref[...] = (jnp.dot(h2, wh_ref[...],
                              preferred_element_type=jnp.float32)
                      + bh_ref[...])


def _tc_mlp(x, W1, b1r, W2, b2r, Wh, bh):
    grid = (1, _D_IN // _TK)
    return pl.pallas_call(
        _mlp_kernel,
        grid=grid,
        in_specs=[
            pl.BlockSpec((_M, _TK), lambda m, k: (m, k)),
            pl.BlockSpec((_TK, _D_HID), lambda m, k: (k, 0)),
            pl.BlockSpec((1, _D_HID), lambda m, k: (0, 0)),
            pl.BlockSpec((_D_HID, _D_HID), lambda m, k: (0, 0)),
            pl.BlockSpec((1, _D_HID), lambda m, k: (0, 0)),
            pl.BlockSpec((_D_HID, _OUT), lambda m, k: (0, 0)),
            pl.BlockSpec((1, _OUT), lambda m, k: (0, 0)),
        ],
        out_specs=pl.BlockSpec((_M, _OUT), lambda m, k: (m, 0)),
        out_shape=jax.ShapeDtypeStruct((_M, _OUT), jnp.float32),
        scratch_shapes=[pltpu.VMEM((_M, _D_HID), jnp.float32)],
        compiler_params=pltpu.CompilerParams(
            dimension_semantics=("parallel", "arbitrary")),
    )(x, W1, b1r, W2, b2r, Wh, bh)


def kernel(aligned_feat, W1, b1, W2, b2, Wc, bc, Wr, br):
    Wh = jnp.concatenate([Wc, Wr], axis=1)
    bh = jnp.concatenate([bc, br]).reshape(1, _OUT)
    b1r = b1.reshape(1, _D_HID)
    b2r = b2.reshape(1, _D_HID)

    x4 = _sc_compact(aligned_feat)
    x = x4.reshape(_M, _D_IN)
    out = _tc_mlp(x, W1, b1r, W2, b2r, Wh, bh)
    return out.reshape(_B, _N, _OUT)


# speedup vs baseline: 15.1457x; 15.1457x over previous
import jax
import jax.numpy as jnp
from jax.experimental import pallas as pl
from jax.experimental.pallas import tpu as pltpu

_B, _N, _C, _H, _W = 2, 512, 256, 7, 7
_D_IN = _C * _H * _W
_D_HID = 1024
_NCLS = 11
_NREG = 5
_OUT = _NCLS + _NREG
_M = _B * _N

_TM = 1024
_TK = 1792


def _mlp_kernel(x_ref, w1_ref, b1_ref, w2_ref, b2_ref, wc_ref, bc_ref,
                wr_ref, br_ref, o_ref, acc_ref):
    k = pl.program_id(1)

    @pl.when(k == 0)
    def _init():
        acc_ref[...] = jnp.zeros_like(acc_ref)

    acc_ref[...] += jnp.dot(x_ref[...], w1_ref[...],
                            preferred_element_type=jnp.float32)

    @pl.when(k == pl.num_programs(1) - 1)
    def _finish():
        h1 = jnp.maximum(acc_ref[...] + b1_ref[...][None, :], 0.0)
        h2 = jnp.maximum(
            jnp.dot(h1, w2_ref[...], preferred_element_type=jnp.float32)
            + b2_ref[...][None, :], 0.0)
        cls = (jnp.dot(h2, wc_ref[...], preferred_element_type=jnp.float32)
               + bc_ref[...][None, :])
        reg = (jnp.dot(h2, wr_ref[...], preferred_element_type=jnp.float32)
               + br_ref[...][None, :])
        o_ref[:, :_NCLS] = cls
        o_ref[:, _NCLS:] = reg


def kernel(aligned_feat, W1, b1, W2, b2, Wc, bc, Wr, br):
    x = aligned_feat.reshape(_M, _D_IN)

    grid = (_M // _TM, _D_IN // _TK)
    out = pl.pallas_call(
        _mlp_kernel,
        grid=grid,
        in_specs=[
            pl.BlockSpec((_TM, _TK), lambda m, k: (m, k)),
            pl.BlockSpec((_TK, _D_HID), lambda m, k: (k, 0)),
            pl.BlockSpec((_D_HID,), lambda m, k: (0,)),
            pl.BlockSpec((_D_HID, _D_HID), lambda m, k: (0, 0)),
            pl.BlockSpec((_D_HID,), lambda m, k: (0,)),
            pl.BlockSpec((_D_HID, _NCLS), lambda m, k: (0, 0)),
            pl.BlockSpec((_NCLS,), lambda m, k: (0,)),
            pl.BlockSpec((_D_HID, _NREG), lambda m, k: (0, 0)),
            pl.BlockSpec((_NREG,), lambda m, k: (0,)),
        ],
        out_specs=pl.BlockSpec((_TM, _OUT), lambda m, k: (m, 0)),
        out_shape=jax.ShapeDtypeStruct((_M, _OUT), jnp.float32),
        scratch_shapes=[pltpu.VMEM((_TM, _D_HID), jnp.float32)],
        compiler_params=pltpu.CompilerParams(
            dimension_semantics=("parallel", "arbitrary")),
    )(x, W1, b1, W2, b2, Wc, bc, Wr, br)
    return out.reshape(_B, _N, _OUT)
